# Initial kernel scaffold; baseline (speedup 1.0000x reference)
import jax
import jax.numpy as jnp
from jax.experimental import pallas as pl


def kernel(feature, edge_index, edge_type, W1, q1, k1, b1, W2, q2, k2, b2):
    raise NotImplementedError("write your pallas kernel here")


# speedup vs baseline: 31.8809x; 31.8809x over previous
import functools

import jax
import jax.numpy as jnp
from jax import lax
from jax.experimental import pallas as pl
from jax.experimental.pallas import tpu as pltpu
from jax.experimental.pallas import tpu_sc as plsc

N = 50000
E = 800000
IN_DIM = 16
HID = 64
OUT = 3

NC = 2
NS = 16
NW = NC * NS
EPT = 25600
E_PAD = NW * EPT
CH = 512
NCHUNK = EPT // CH
NVREG = CH // 16

L1_ROWS = 2 * N + 96
L2_ROWS = N + 48
ZR = 136


def _edge_pass(acc_rows, gather_by_ik):
    rpt = acc_rows // NS
    mesh = plsc.VectorSubcoreMesh(core_axis_name="c", subcore_axis_name="s")

    @functools.partial(
        pl.kernel,
        mesh=mesh,
        compiler_params=pltpu.CompilerParams(use_tc_tiling_on_sc=False),
        out_type=[
            jax.ShapeDtypeStruct((NC, acc_rows, 16), jnp.float32),
            jax.ShapeDtypeStruct((NC * acc_rows,), jnp.float32),
        ],
        scratch_types=[
            pltpu.VMEM_SHARED((acc_rows, 16), jnp.float32),
            pltpu.VMEM_SHARED((acc_rows,), jnp.float32),
            pltpu.VMEM((ZR, 16), jnp.float32),
            pltpu.VMEM((CH,), jnp.float32),
            pltpu.VMEM((CH,), jnp.int32),
            pltpu.VMEM((CH,), jnp.int32),
            pltpu.VMEM((CH,), jnp.int32),
            pltpu.VMEM((CH,), jnp.int32),
            pltpu.VMEM((CH,), jnp.int32),
            pltpu.VMEM((CH,), jnp.int32),
            pltpu.VMEM((CH,), jnp.float32),
            pltpu.VMEM((CH,), jnp.float32),
            pltpu.VMEM((CH,), jnp.float32),
            pltpu.VMEM((CH, 16), jnp.float32),
            pltpu.SemaphoreType.DMA,
            pltpu.SemaphoreType.DMA,
        ],
    )
    def kern(sqf, skf, tab, srcv, dstv, etv, agg_out, den_out,
             agg_s, den_s, zbuf, zflat, sbuf, dbuf, tbuf, iqb, ikb, giqb,
             sqv, skv, ab, rows, sem, sem2):
        c = lax.axis_index("c")
        s = lax.axis_index("s")
        wid = c * NS + s
        tbase = wid * EPT
        rbase = s * rpt
        gmax = 2 * N - 1

        def zb(i, carry):
            zbuf[i, :] = jnp.zeros((16,), jnp.float32)
            return carry
        lax.fori_loop(0, ZR, zb, 0)

        def zf(i, carry):
            zflat[pl.ds(i * 16, 16)] = jnp.zeros((16,), jnp.float32)
            return carry
        lax.fori_loop(0, NVREG, zf, 0)

        for j in range(rpt // ZR):
            pltpu.sync_copy(zbuf, agg_s.at[pl.ds(rbase + j * ZR, ZR)])
        for j in range(rpt // CH):
            pltpu.sync_copy(zflat, den_s.at[pl.ds(rbase + j * CH, CH)])
        rem = rpt % CH
        if rem:
            pltpu.sync_copy(zflat.at[pl.ds(0, rem)],
                            den_s.at[pl.ds(rbase + (rpt // CH) * CH, rem)])
        plsc.subcore_barrier()

        def chunk(i, carry):
            eb = tbase + i * CH
            pltpu.sync_copy(srcv.at[pl.ds(eb, CH)], sbuf)
            pltpu.sync_copy(dstv.at[pl.ds(eb, CH)], dbuf)
            pltpu.sync_copy(etv.at[pl.ds(eb, CH)], tbuf)

            def vix(jv, cc):
                sl = pl.ds(jv * 16, 16)
                d = dbuf[sl]
                t = tbuf[sl]
                sr = sbuf[sl]
                iq = 2 * d + t
                iqb[sl] = iq
                ikb[sl] = 2 * sr + t
                giqb[sl] = jnp.minimum(iq, gmax)
                return cc
            lax.fori_loop(0, NVREG, vix, 0)

            pltpu.async_copy(sqf.at[giqb], sqv, sem).wait()
            pltpu.async_copy(skf.at[ikb], skv, sem).wait()
            if gather_by_ik:
                pltpu.async_copy(tab.at[ikb], rows, sem2).wait()
            else:
                pltpu.async_copy(tab.at[sbuf], rows, sem2).wait()

            def av(jv, cc):
                sl = pl.ds(jv * 16, 16)
                l = sqv[sl] + skv[sl]
                l = jnp.where(l >= 0.0, l, 0.2 * l)
                ab[sl] = jnp.exp(l)
                return cc
            lax.fori_loop(0, NVREG, av, 0)

            def scale(jv, cc):
                a16 = ab[pl.ds(jv * 16, 16)]
                for l in range(16):
                    rows[jv * 16 + l, :] = rows[jv * 16 + l, :] * a16[l]
                return cc
            lax.fori_loop(0, NVREG, scale, 0)

            if gather_by_ik:
                pltpu.sync_copy(rows, agg_s.at[dbuf], add=True)
                pltpu.sync_copy(ab, den_s.at[dbuf], add=True)
            else:
                pltpu.sync_copy(rows, agg_s.at[iqb], add=True)
                pltpu.sync_copy(ab, den_s.at[iqb], add=True)
            return carry
        lax.fori_loop(0, NCHUNK, chunk, 0)
        plsc.subcore_barrier()

        for j in range(rpt // ZR):
            sl = pl.ds(rbase + j * ZR, ZR)
            pltpu.sync_copy(agg_s.at[sl], agg_out.at[c, sl])
        dbase = c * acc_rows + rbase
        for j in range(rpt // CH):
            pltpu.sync_copy(den_s.at[pl.ds(rbase + j * CH, CH)],
                            den_out.at[pl.ds(dbase + j * CH, CH)])
        if rem:
            pltpu.sync_copy(den_s.at[pl.ds(rbase + (rpt // CH) * CH, rem)],
                            den_out.at[pl.ds(dbase + (rpt // CH) * CH, rem)])

    return kern


_edge_pass_l1 = _edge_pass(L1_ROWS, gather_by_ik=False)
_edge_pass_l2 = _edge_pass(L2_ROWS, gather_by_ik=True)

_B = 2000
_GRID = N // _B


def _full(shape):
    return pl.BlockSpec(shape, lambda i: (0,) * len(shape))


def _pre_kernel(x_ref, w1_ref, q1_ref, k1_ref, sq_ref, sk_ref):
    aq = jnp.concatenate([(w1_ref[0] @ q1_ref[0])[:, None],
                          (w1_ref[1] @ q1_ref[0])[:, None]], axis=1)
    ak = jnp.concatenate([(w1_ref[0] @ k1_ref[0])[:, None],
                          (w1_ref[1] @ k1_ref[0])[:, None]], axis=1)
    sq_ref[...] = x_ref[...] @ aq
    sk_ref[...] = x_ref[...] @ ak


def _mid_kernel(agg_ref, den_ref, w1_ref, b1_ref, w2_ref, q2_ref, k2_ref,
                xt2_ref, sq2_ref, sk2_ref):
    a = agg_ref[0] + agg_ref[1]
    h = a[:, 0, :] @ w1_ref[0] + a[:, 1, :] @ w1_ref[1]
    d = jnp.sum(den_ref[...], axis=(0, 2))
    h = h / (d[:, None] + 1e-16) + b1_ref[0][None, :]
    xt0 = h @ w2_ref[0]
    xt1 = h @ w2_ref[1]
    z = jnp.zeros((_B, 1, 16 - OUT), jnp.float32)
    xt2_ref[...] = jnp.concatenate(
        [xt0[:, None, :], z, xt1[:, None, :], z], axis=-1).reshape(_B, 2, 16)
    aq2 = jnp.concatenate([(w2_ref[0] @ q2_ref[0])[:, None],
                           (w2_ref[1] @ q2_ref[0])[:, None]], axis=1)
    ak2 = jnp.concatenate([(w2_ref[0] @ k2_ref[0])[:, None],
                           (w2_ref[1] @ k2_ref[0])[:, None]], axis=1)
    sq2_ref[...] = h @ aq2
    sk2_ref[...] = h @ ak2


def _post_kernel(acc_ref, den_ref, b2_ref, out_ref):
    a = acc_ref[0] + acc_ref[1]
    d = jnp.sum(den_ref[...], axis=(0, 2))
    out_ref[...] = a[:, :OUT] / (d[:, None] + 1e-16) + b2_ref[0][None, :]


def kernel(feature, edge_index, edge_type, W1, q1, k1, b1, W2, q2, k2, b2):
    src = edge_index[0]
    dst = edge_index[1]
    npad = E_PAD - E
    src_p = jnp.concatenate([src, jnp.zeros((npad,), jnp.int32)])
    dst_p = jnp.concatenate([dst, jnp.full((npad,), N, jnp.int32)])
    et_p = jnp.concatenate([edge_type, jnp.zeros((npad,), jnp.int32)])

    q1r = q1.reshape(1, HID)
    k1r = k1.reshape(1, HID)
    b1r = b1.reshape(1, HID)
    q2r = q2.reshape(1, OUT)
    k2r = k2.reshape(1, OUT)
    b2r = b2.reshape(1, OUT)

    sq1, sk1 = pl.pallas_call(
        _pre_kernel,
        grid=(_GRID,),
        in_specs=[
            pl.BlockSpec((_B, IN_DIM), lambda i: (i, 0)),
            _full((2, IN_DIM, HID)),
            _full((1, HID)),
            _full((1, HID)),
        ],
        out_specs=[
            pl.BlockSpec((_B, 2), lambda i: (i, 0)),
            pl.BlockSpec((_B, 2), lambda i: (i, 0)),
        ],
        out_shape=[
            jax.ShapeDtypeStruct((N, 2), jnp.float32),
            jax.ShapeDtypeStruct((N, 2), jnp.float32),
        ],
    )(feature, W1, q1r, k1r)

    agg1, den1 = _edge_pass_l1(
        sq1.reshape(2 * N), sk1.reshape(2 * N), feature, src_p, dst_p, et_p)

    agg1n = agg1[:, :2 * N].reshape(2, N, 2, 16)
    den1n = den1.reshape(2, L1_ROWS)[:, :2 * N].reshape(2, N, 2)

    xt2p, sq2, sk2 = pl.pallas_call(
        _mid_kernel,
        grid=(_GRID,),
        in_specs=[
            pl.BlockSpec((2, _B, 2, 16), lambda i: (0, i, 0, 0)),
            pl.BlockSpec((2, _B, 2), lambda i: (0, i, 0)),
            _full((2, IN_DIM, HID)),
            _full((1, HID)),
            _full((2, HID, OUT)),
            _full((1, OUT)),
            _full((1, OUT)),
        ],
        out_specs=[
            pl.BlockSpec((_B, 2, 16), lambda i: (i, 0, 0)),
            pl.BlockSpec((_B, 2), lambda i: (i, 0)),
            pl.BlockSpec((_B, 2), lambda i: (i, 0)),
        ],
        out_shape=[
            jax.ShapeDtypeStruct((N, 2, 16), jnp.float32),
            jax.ShapeDtypeStruct((N, 2), jnp.float32),
            jax.ShapeDtypeStruct((N, 2), jnp.float32),
        ],
    )(agg1n, den1n, W1, b1r, W2, q2r, k2r)

    acc2, den2 = _edge_pass_l2(
        sq2.reshape(2 * N), sk2.reshape(2 * N), xt2p.reshape(2 * N, 16),
        src_p, dst_p, et_p)

    acc2n = acc2[:, :N]
    den2n = den2.reshape(2, L2_ROWS)[:, :N].reshape(2, N, 1)

    out = pl.pallas_call(
        _post_kernel,
        grid=(_GRID,),
        in_specs=[
            pl.BlockSpec((2, _B, 16), lambda i: (0, i, 0)),
            pl.BlockSpec((2, _B, 1), lambda i: (0, i, 0)),
            _full((1, OUT)),
        ],
        out_specs=pl.BlockSpec((_B, OUT), lambda i: (i, 0)),
        out_shape=jax.ShapeDtypeStruct((N, OUT), jnp.float32),
    )(acc2n, den2n, b2r)

    return out


# speedup vs baseline: 40.1107x; 1.2581x over previous
import functools

import jax
import jax.numpy as jnp
from jax import lax
from jax.experimental import pallas as pl
from jax.experimental.pallas import tpu as pltpu
from jax.experimental.pallas import tpu_sc as plsc

N = 50000
E = 800000
IN_DIM = 16
HID = 64
OUT = 3

NC = 2
NS = 16
NW = NC * NS
EPT = 25600
E_PAD = NW * EPT
L1_ROWS = 2 * N + 96
L2_ROWS = N + 48
ZR = 136


def _edge_pass(acc_rows, gather_by_ik, ch):
    rpt = acc_rows // NS
    nchunk = EPT // ch
    nvreg = ch // 16
    mesh = plsc.VectorSubcoreMesh(core_axis_name="c", subcore_axis_name="s")

    @functools.partial(
        pl.kernel,
        mesh=mesh,
        compiler_params=pltpu.CompilerParams(use_tc_tiling_on_sc=False),
        out_type=[
            jax.ShapeDtypeStruct((NC, acc_rows, 16), jnp.float32),
            jax.ShapeDtypeStruct((NC * acc_rows,), jnp.float32),
        ],
        scratch_types=[
            pltpu.VMEM_SHARED((acc_rows, 16), jnp.float32),
            pltpu.VMEM_SHARED((acc_rows,), jnp.float32),
            pltpu.VMEM((ZR, 16), jnp.float32),
            pltpu.VMEM((3, ch), jnp.int32),
            pltpu.VMEM((2 * ch,), jnp.int32),
            pltpu.VMEM((ch,), jnp.int32),
            pltpu.VMEM((ch,), jnp.int32),
            pltpu.VMEM((2 * ch,), jnp.float32),
            pltpu.VMEM((ch,), jnp.float32),
            pltpu.VMEM((ch, 16), jnp.float32),
            pltpu.SemaphoreType.DMA,
            pltpu.SemaphoreType.DMA,
        ],
    )
    def kern(qkt, tab, e3, agg_out, den_out,
             agg_s, den_s, zbuf, ebuf, ckb, gidx, sidx, qkv, ab, rows,
             sem, sem2):
        c = lax.axis_index("c")
        s = lax.axis_index("s")
        wid = c * NS + s
        tbase = wid * EPT
        rbase = s * rpt
        gmax = 2 * N - 1

        def zb(i, carry):
            zbuf[i, :] = jnp.zeros((16,), jnp.float32)
            return carry
        lax.fori_loop(0, ZR, zb, 0)

        def zf(i, carry):
            ab[pl.ds(i * 16, 16)] = jnp.zeros((16,), jnp.float32)
            return carry
        lax.fori_loop(0, nvreg, zf, 0)

        for j in range(rpt // ZR):
            pltpu.sync_copy(zbuf, agg_s.at[pl.ds(rbase + j * ZR, ZR)])
        for j in range(rpt // ch):
            pltpu.sync_copy(ab, den_s.at[pl.ds(rbase + j * ch, ch)])
        rem = rpt % ch
        if rem:
            pltpu.sync_copy(ab.at[pl.ds(0, rem)],
                            den_s.at[pl.ds(rbase + (rpt // ch) * ch, rem)])
        plsc.subcore_barrier()

        def chunk(i, carry):
            eb = tbase + i * ch
            pltpu.sync_copy(e3.at[:, pl.ds(eb, ch)], ebuf)

            def vix(jv, cc):
                sl = pl.ds(jv * 16, 16)
                sr = ebuf[0, sl]
                d = ebuf[1, sl]
                t = ebuf[2, sl]
                iq = 2 * d + t
                ik = 2 * sr + t
                ckb[sl] = jnp.minimum(iq, gmax)
                ckb[pl.ds(ch + jv * 16, 16)] = 2 * N + ik
                if gather_by_ik:
                    gidx[sl] = ik
                    sidx[sl] = d
                else:
                    gidx[sl] = sr
                    sidx[sl] = iq
                return cc
            lax.fori_loop(0, nvreg, vix, 0)

            cpr = pltpu.async_copy(tab.at[gidx], rows, sem2)
            cpq = pltpu.async_copy(qkt.at[ckb], qkv, sem)
            cpq.wait()

            def av(jv, cc):
                sl = pl.ds(jv * 16, 16)
                l = qkv[sl] + qkv[pl.ds(ch + jv * 16, 16)]
                l = jnp.where(l >= 0.0, l, 0.2 * l)
                ab[sl] = jnp.exp(l)
                return cc
            lax.fori_loop(0, nvreg, av, 0)
            cpr.wait()

            def scale(jv, cc):
                a16 = ab[pl.ds(jv * 16, 16)]
                for l in range(16):
                    rows[jv * 16 + l, :] = rows[jv * 16 + l, :] * a16[l]
                return cc
            lax.fori_loop(0, nvreg, scale, 0)

            pltpu.sync_copy(rows, agg_s.at[sidx], add=True)
            pltpu.sync_copy(ab, den_s.at[sidx], add=True)
            return carry
        lax.fori_loop(0, nchunk, chunk, 0)
        plsc.subcore_barrier()

        for j in range(rpt // ZR):
            sl = pl.ds(rbase + j * ZR, ZR)
            pltpu.sync_copy(agg_s.at[sl], agg_out.at[c, sl])
        dbase = c * acc_rows + rbase
        pltpu.sync_copy(den_s.at[pl.ds(rbase, rpt)],
                        den_out.at[pl.ds(dbase, rpt)])

    return kern


_edge_pass_l1 = _edge_pass(L1_ROWS, gather_by_ik=False, ch=640)
_edge_pass_l2 = _edge_pass(L2_ROWS, gather_by_ik=True, ch=1024)

_B = 2000
_GRID = N // _B


def _full(shape):
    return pl.BlockSpec(shape, lambda i: (0,) * len(shape))


def _pre_kernel(x_ref, w1_ref, q1_ref, k1_ref, sq_ref, sk_ref):
    aq = jnp.concatenate([(w1_ref[0] @ q1_ref[0])[:, None],
                          (w1_ref[1] @ q1_ref[0])[:, None]], axis=1)
    ak = jnp.concatenate([(w1_ref[0] @ k1_ref[0])[:, None],
                          (w1_ref[1] @ k1_ref[0])[:, None]], axis=1)
    sq_ref[...] = x_ref[...] @ aq
    sk_ref[...] = x_ref[...] @ ak


def _mid_kernel(agg_ref, den_ref, w1_ref, b1_ref, w2_ref, q2_ref, k2_ref,
                xt2_ref, sq2_ref, sk2_ref):
    a = agg_ref[0] + agg_ref[1]
    h = a[:, 0, :] @ w1_ref[0] + a[:, 1, :] @ w1_ref[1]
    d = jnp.sum(den_ref[...], axis=(0, 2))
    h = h / (d[:, None] + 1e-16) + b1_ref[0][None, :]
    xt0 = h @ w2_ref[0]
    xt1 = h @ w2_ref[1]
    z = jnp.zeros((_B, 1, 16 - OUT), jnp.float32)
    xt2_ref[...] = jnp.concatenate(
        [xt0[:, None, :], z, xt1[:, None, :], z], axis=-1).reshape(_B, 2, 16)
    aq2 = jnp.concatenate([(w2_ref[0] @ q2_ref[0])[:, None],
                           (w2_ref[1] @ q2_ref[0])[:, None]], axis=1)
    ak2 = jnp.concatenate([(w2_ref[0] @ k2_ref[0])[:, None],
                           (w2_ref[1] @ k2_ref[0])[:, None]], axis=1)
    sq2_ref[...] = h @ aq2
    sk2_ref[...] = h @ ak2


def _post_kernel(acc_ref, den_ref, b2_ref, out_ref):
    a = acc_ref[0] + acc_ref[1]
    d = jnp.sum(den_ref[...], axis=(0, 2))
    out_ref[...] = a[:, :OUT] / (d[:, None] + 1e-16) + b2_ref[0][None, :]


def kernel(feature, edge_index, edge_type, W1, q1, k1, b1, W2, q2, k2, b2):
    src = edge_index[0]
    dst = edge_index[1]
    npad = E_PAD - E
    src_p = jnp.concatenate([src, jnp.zeros((npad,), jnp.int32)])
    dst_p = jnp.concatenate([dst, jnp.full((npad,), N, jnp.int32)])
    et_p = jnp.concatenate([edge_type, jnp.zeros((npad,), jnp.int32)])

    q1r = q1.reshape(1, HID)
    k1r = k1.reshape(1, HID)
    b1r = b1.reshape(1, HID)
    q2r = q2.reshape(1, OUT)
    k2r = k2.reshape(1, OUT)
    b2r = b2.reshape(1, OUT)

    sq1, sk1 = pl.pallas_call(
        _pre_kernel,
        grid=(_GRID,),
        in_specs=[
            pl.BlockSpec((_B, IN_DIM), lambda i: (i, 0)),
            _full((2, IN_DIM, HID)),
            _full((1, HID)),
            _full((1, HID)),
        ],
        out_specs=[
            pl.BlockSpec((_B, 2), lambda i: (i, 0)),
            pl.BlockSpec((_B, 2), lambda i: (i, 0)),
        ],
        out_shape=[
            jax.ShapeDtypeStruct((N, 2), jnp.float32),
            jax.ShapeDtypeStruct((N, 2), jnp.float32),
        ],
    )(feature, W1, q1r, k1r)

    e3 = jnp.stack([src_p, dst_p, et_p])
    qk1 = jnp.concatenate([sq1.reshape(2 * N), sk1.reshape(2 * N)])
    agg1, den1 = _edge_pass_l1(qk1, feature, e3)

    agg1n = agg1[:, :2 * N].reshape(2, N, 2, 16)
    den1n = den1.reshape(2, L1_ROWS)[:, :2 * N].reshape(2, N, 2)

    xt2p, sq2, sk2 = pl.pallas_call(
        _mid_kernel,
        grid=(_GRID,),
        in_specs=[
            pl.BlockSpec((2, _B, 2, 16), lambda i: (0, i, 0, 0)),
            pl.BlockSpec((2, _B, 2), lambda i: (0, i, 0)),
            _full((2, IN_DIM, HID)),
            _full((1, HID)),
            _full((2, HID, OUT)),
            _full((1, OUT)),
            _full((1, OUT)),
        ],
        out_specs=[
            pl.BlockSpec((_B, 2, 16), lambda i: (i, 0, 0)),
            pl.BlockSpec((_B, 2), lambda i: (i, 0)),
            pl.BlockSpec((_B, 2), lambda i: (i, 0)),
        ],
        out_shape=[
            jax.ShapeDtypeStruct((N, 2, 16), jnp.float32),
            jax.ShapeDtypeStruct((N, 2), jnp.float32),
            jax.ShapeDtypeStruct((N, 2), jnp.float32),
        ],
    )(agg1n, den1n, W1, b1r, W2, q2r, k2r)

    qk2 = jnp.concatenate([sq2.reshape(2 * N), sk2.reshape(2 * N)])
    acc2, den2 = _edge_pass_l2(qk2, xt2p.reshape(2 * N, 16), e3)

    acc2n = acc2[:, :N]
    den2n = den2.reshape(2, L2_ROWS)[:, :N].reshape(2, N, 1)

    out = pl.pallas_call(
        _post_kernel,
        grid=(_GRID,),
        in_specs=[
            pl.BlockSpec((2, _B, 16), lambda i: (0, i, 0)),
            pl.BlockSpec((2, _B, 1), lambda i: (0, i, 0)),
            _full((1, OUT)),
        ],
        out_specs=pl.BlockSpec((_B, OUT), lambda i: (i, 0)),
        out_shape=jax.ShapeDtypeStruct((N, OUT), jnp.float32),
    )(acc2n, den2n, b2r)

    return out


# speedup vs baseline: 44.7351x; 1.1153x over previous
import functools

import jax
import jax.numpy as jnp
from jax import lax
from jax.experimental import pallas as pl
from jax.experimental.pallas import tpu as pltpu
from jax.experimental.pallas import tpu_sc as plsc

N = 50000
E = 800000
IN_DIM = 16
HID = 64
OUT = 3

NC = 2
NS = 16
NW = NC * NS
EPT = 25600
E_PAD = NW * EPT
L1_ROWS = 2 * N + 96
L2_ROWS = N + 48
ZR = 136


def _edge_pass(acc_rows, gather_by_ik, ch):
    rpt = acc_rows // NS
    nchunk = EPT // ch
    nvreg = ch // 16
    mesh = plsc.VectorSubcoreMesh(core_axis_name="c", subcore_axis_name="s")

    nbuf = 2
    buf_types = []
    for _ in range(nbuf):
        buf_types += [
            pltpu.VMEM((3, ch), jnp.int32),
            pltpu.VMEM((2 * ch,), jnp.int32),
            pltpu.VMEM((ch,), jnp.int32),
            pltpu.VMEM((ch,), jnp.int32),
            pltpu.VMEM((2 * ch,), jnp.float32),
            pltpu.VMEM((ch,), jnp.float32),
            pltpu.VMEM((ch, 16), jnp.float32),
            pltpu.SemaphoreType.DMA,
            pltpu.SemaphoreType.DMA,
            pltpu.SemaphoreType.DMA,
            pltpu.SemaphoreType.DMA,
        ]

    @functools.partial(
        pl.kernel,
        mesh=mesh,
        compiler_params=pltpu.CompilerParams(use_tc_tiling_on_sc=False),
        out_type=[
            jax.ShapeDtypeStruct((NC, acc_rows, 16), jnp.float32),
            jax.ShapeDtypeStruct((NC * acc_rows,), jnp.float32),
        ],
        scratch_types=[
            pltpu.VMEM_SHARED((acc_rows, 16), jnp.float32),
            pltpu.VMEM_SHARED((acc_rows,), jnp.float32),
            pltpu.VMEM((ZR, 16), jnp.float32),
        ] + buf_types,
    )
    def kern(qkt, tab, e3, agg_out, den_out, agg_s, den_s, zbuf, *bufs):
        A = dict(zip(
            ("ebuf", "ckb", "gidx", "sidx", "qkv", "ab", "rows",
             "semE", "semQ", "semR", "semS"), bufs[:11]))
        B = dict(zip(
            ("ebuf", "ckb", "gidx", "sidx", "qkv", "ab", "rows",
             "semE", "semQ", "semR", "semS"), bufs[11:]))
        c = lax.axis_index("c")
        s = lax.axis_index("s")
        wid = c * NS + s
        tbase = wid * EPT
        rbase = s * rpt
        gmax = 2 * N - 1

        def fire_idx(i, st):
            pltpu.async_copy(e3.at[:, pl.ds(tbase + i * ch, ch)],
                             st["ebuf"], st["semE"])

        def wait_idx(st):
            pltpu.make_async_copy(e3.at[:, pl.ds(tbase, ch)],
                                  st["ebuf"], st["semE"]).wait()

        def vix(st):
            ebuf, ckb, gidx, sidx = st["ebuf"], st["ckb"], st["gidx"], st["sidx"]
            def body(jv, cc):
                sl = pl.ds(jv * 16, 16)
                sr = ebuf[0, sl]
                d = ebuf[1, sl]
                t = ebuf[2, sl]
                iq = 2 * d + t
                ik = 2 * sr + t
                ckb[sl] = jnp.minimum(iq, gmax)
                ckb[pl.ds(ch + jv * 16, 16)] = 2 * N + ik
                if gather_by_ik:
                    gidx[sl] = ik
                    sidx[sl] = d
                else:
                    gidx[sl] = sr
                    sidx[sl] = iq
                return cc
            lax.fori_loop(0, nvreg, body, 0)

        def fire_gathers(st):
            pltpu.async_copy(tab.at[st["gidx"]], st["rows"], st["semR"])
            pltpu.async_copy(qkt.at[st["ckb"]], st["qkv"], st["semQ"])

        def wait_scatter(st):
            pltpu.make_async_copy(st["rows"], agg_s.at[st["sidx"]],
                                  st["semS"]).wait()
            pltpu.make_async_copy(st["ab"], den_s.at[st["sidx"]],
                                  st["semS"]).wait()

        def compute_and_scatter(st):
            qkv, ab, rows = st["qkv"], st["ab"], st["rows"]
            pltpu.make_async_copy(qkt.at[st["ckb"]], qkv, st["semQ"]).wait()

            def av(jv, cc):
                sl = pl.ds(jv * 16, 16)
                l = qkv[sl] + qkv[pl.ds(ch + jv * 16, 16)]
                l = jnp.where(l >= 0.0, l, 0.2 * l)
                ab[sl] = jnp.exp(l)
                return cc
            lax.fori_loop(0, nvreg, av, 0)
            pltpu.make_async_copy(tab.at[st["gidx"]], rows, st["semR"]).wait()

            def scale(jv, cc):
                a16 = ab[pl.ds(jv * 16, 16)]
                for l in range(16):
                    rows[jv * 16 + l, :] = rows[jv * 16 + l, :] * a16[l]
                return cc
            lax.fori_loop(0, nvreg, scale, 0)

            pltpu.async_copy(rows, agg_s.at[st["sidx"]], st["semS"], add=True)
            pltpu.async_copy(ab, den_s.at[st["sidx"]], st["semS"], add=True)

        def zb(i, carry):
            zbuf[i, :] = jnp.zeros((16,), jnp.float32)
            return carry
        lax.fori_loop(0, ZR, zb, 0)

        def zf(i, carry):
            A["ab"][pl.ds(i * 16, 16)] = jnp.zeros((16,), jnp.float32)
            return carry
        lax.fori_loop(0, nvreg, zf, 0)

        for j in range(rpt // ZR):
            pltpu.sync_copy(zbuf, agg_s.at[pl.ds(rbase + j * ZR, ZR)])
        for j in range(rpt // ch):
            pltpu.sync_copy(A["ab"], den_s.at[pl.ds(rbase + j * ch, ch)])
        rem = rpt % ch
        if rem:
            pltpu.sync_copy(A["ab"].at[pl.ds(0, rem)],
                            den_s.at[pl.ds(rbase + (rpt // ch) * ch, rem)])
        plsc.subcore_barrier()

        fire_idx(0, A)
        wait_idx(A)
        vix(A)
        fire_gathers(A)
        fire_idx(1, B)

        def body(big, carry):
            i0 = 2 * big
            wait_idx(B)

            @pl.when(big > 0)
            def _():
                wait_scatter(B)
            vix(B)
            fire_gathers(B)

            @pl.when(i0 + 2 < nchunk)
            def _():
                fire_idx(i0 + 2, A)
            compute_and_scatter(A)
            @pl.when(i0 + 2 < nchunk)
            def _():
                wait_idx(A)
                wait_scatter(A)
                vix(A)
                fire_gathers(A)
                fire_idx(i0 + 3, B)
            compute_and_scatter(B)
            return carry
        lax.fori_loop(0, nchunk // 2, body, 0)
        wait_scatter(A)
        wait_scatter(B)
        plsc.subcore_barrier()

        for j in range(rpt // ZR):
            sl = pl.ds(rbase + j * ZR, ZR)
            pltpu.sync_copy(agg_s.at[sl], agg_out.at[c, sl])
        dbase = c * acc_rows + rbase
        pltpu.sync_copy(den_s.at[pl.ds(rbase, rpt)],
                        den_out.at[pl.ds(dbase, rpt)])

    return kern


_edge_pass_l1 = _edge_pass(L1_ROWS, gather_by_ik=False, ch=400)
_edge_pass_l2 = _edge_pass(L2_ROWS, gather_by_ik=True, ch=800)

_B = 2000
_GRID = N // _B


def _full(shape):
    return pl.BlockSpec(shape, lambda i: (0,) * len(shape))


def _pre_kernel(x_ref, w1_ref, q1_ref, k1_ref, sq_ref, sk_ref):
    aq = jnp.concatenate([(w1_ref[0] @ q1_ref[0])[:, None],
                          (w1_ref[1] @ q1_ref[0])[:, None]], axis=1)
    ak = jnp.concatenate([(w1_ref[0] @ k1_ref[0])[:, None],
                          (w1_ref[1] @ k1_ref[0])[:, None]], axis=1)
    sq_ref[...] = x_ref[...] @ aq
    sk_ref[...] = x_ref[...] @ ak


def _mid_kernel(agg_ref, den_ref, w1_ref, b1_ref, w2_ref, q2_ref, k2_ref,
                xt2_ref, sq2_ref, sk2_ref):
    a = agg_ref[0] + agg_ref[1]
    h = a[:, 0, :] @ w1_ref[0] + a[:, 1, :] @ w1_ref[1]
    d = jnp.sum(den_ref[...], axis=(0, 2))
    h = h / (d[:, None] + 1e-16) + b1_ref[0][None, :]
    xt0 = h @ w2_ref[0]
    xt1 = h @ w2_ref[1]
    z = jnp.zeros((_B, 1, 16 - OUT), jnp.float32)
    xt2_ref[...] = jnp.concatenate(
        [xt0[:, None, :], z, xt1[:, None, :], z], axis=-1).reshape(_B, 2, 16)
    aq2 = jnp.concatenate([(w2_ref[0] @ q2_ref[0])[:, None],
                           (w2_ref[1] @ q2_ref[0])[:, None]], axis=1)
    ak2 = jnp.concatenate([(w2_ref[0] @ k2_ref[0])[:, None],
                           (w2_ref[1] @ k2_ref[0])[:, None]], axis=1)
    sq2_ref[...] = h @ aq2
    sk2_ref[...] = h @ ak2


def _post_kernel(acc_ref, den_ref, b2_ref, out_ref):
    a = acc_ref[0] + acc_ref[1]
    d = jnp.sum(den_ref[...], axis=(0, 2))
    out_ref[...] = a[:, :OUT] / (d[:, None] + 1e-16) + b2_ref[0][None, :]


def kernel(feature, edge_index, edge_type, W1, q1, k1, b1, W2, q2, k2, b2):
    src = edge_index[0]
    dst = edge_index[1]
    npad = E_PAD - E
    src_p = jnp.concatenate([src, jnp.zeros((npad,), jnp.int32)])
    dst_p = jnp.concatenate([dst, jnp.full((npad,), N, jnp.int32)])
    et_p = jnp.concatenate([edge_type, jnp.zeros((npad,), jnp.int32)])

    q1r = q1.reshape(1, HID)
    k1r = k1.reshape(1, HID)
    b1r = b1.reshape(1, HID)
    q2r = q2.reshape(1, OUT)
    k2r = k2.reshape(1, OUT)
    b2r = b2.reshape(1, OUT)

    sq1, sk1 = pl.pallas_call(
        _pre_kernel,
        grid=(_GRID,),
        in_specs=[
            pl.BlockSpec((_B, IN_DIM), lambda i: (i, 0)),
            _full((2, IN_DIM, HID)),
            _full((1, HID)),
            _full((1, HID)),
        ],
        out_specs=[
            pl.BlockSpec((_B, 2), lambda i: (i, 0)),
            pl.BlockSpec((_B, 2), lambda i: (i, 0)),
        ],
        out_shape=[
            jax.ShapeDtypeStruct((N, 2), jnp.float32),
            jax.ShapeDtypeStruct((N, 2), jnp.float32),
        ],
    )(feature, W1, q1r, k1r)

    e3 = jnp.stack([src_p, dst_p, et_p])
    qk1 = jnp.concatenate([sq1.reshape(2 * N), sk1.reshape(2 * N)])
    agg1, den1 = _edge_pass_l1(qk1, feature, e3)

    agg1n = agg1[:, :2 * N].reshape(2, N, 2, 16)
    den1n = den1.reshape(2, L1_ROWS)[:, :2 * N].reshape(2, N, 2)

    xt2p, sq2, sk2 = pl.pallas_call(
        _mid_kernel,
        grid=(_GRID,),
        in_specs=[
            pl.BlockSpec((2, _B, 2, 16), lambda i: (0, i, 0, 0)),
            pl.BlockSpec((2, _B, 2), lambda i: (0, i, 0)),
            _full((2, IN_DIM, HID)),
            _full((1, HID)),
            _full((2, HID, OUT)),
            _full((1, OUT)),
            _full((1, OUT)),
        ],
        out_specs=[
            pl.BlockSpec((_B, 2, 16), lambda i: (i, 0, 0)),
            pl.BlockSpec((_B, 2), lambda i: (i, 0)),
            pl.BlockSpec((_B, 2), lambda i: (i, 0)),
        ],
        out_shape=[
            jax.ShapeDtypeStruct((N, 2, 16), jnp.float32),
            jax.ShapeDtypeStruct((N, 2), jnp.float32),
            jax.ShapeDtypeStruct((N, 2), jnp.float32),
        ],
    )(agg1n, den1n, W1, b1r, W2, q2r, k2r)

    qk2 = jnp.concatenate([sq2.reshape(2 * N), sk2.reshape(2 * N)])
    acc2, den2 = _edge_pass_l2(qk2, xt2p.reshape(2 * N, 16), e3)

    acc2n = acc2[:, :N]
    den2n = den2.reshape(2, L2_ROWS)[:, :N].reshape(2, N, 1)

    out = pl.pallas_call(
        _post_kernel,
        grid=(_GRID,),
        in_specs=[
            pl.BlockSpec((2, _B, 16), lambda i: (0, i, 0)),
            pl.BlockSpec((2, _B, 1), lambda i: (0, i, 0)),
            _full((1, OUT)),
        ],
        out_specs=pl.BlockSpec((_B, OUT), lambda i: (i, 0)),
        out_shape=jax.ShapeDtypeStruct((N, OUT), jnp.float32),
    )(acc2n, den2n, b2r)

    return out
